# baseline (speedup 1.0000x reference)
import dataclasses
import functools

import jax
import jax.numpy as jnp
from jax import lax
from jax.experimental import pallas as pl
from jax.experimental.pallas import tpu as pltpu
from jax.experimental.pallas import tpu_sc as plsc

NN = 10000
NE = 320000
D = 128
NC = 2
NS = 16
NW = NC * NS
EPW = NE // NW
C = 80
CB = 2000
NB = EPW // CB
CPB = CB // C
ZR = 40
NZCH = NN // ZR
ZRB = 8
NZB = NN // ZRB
HR = 80


def _mesh():
    return plsc.VectorSubcoreMesh(core_axis_name="c", subcore_axis_name="s")


def _compiler_params():
    cp = pltpu.CompilerParams()
    if "needs_layout_passes" in pltpu.CompilerParams.__dataclass_fields__:
        cp = dataclasses.replace(cp, needs_layout_passes=False)
    return cp


def _sc_aggregate(h, ei_flat, ev):
    @functools.partial(
        pl.kernel,
        compiler_params=_compiler_params(),
        out_type=[
            jax.ShapeDtypeStruct((NC, NN, D), jnp.float32),
            jax.ShapeDtypeStruct((NC, HR, D), jnp.float32),
        ],
        mesh=_mesh(),
        scratch_types=[
            pltpu.VMEM((CB,), jnp.int32),
            pltpu.VMEM((CB,), jnp.int32),
            pltpu.VMEM((CB,), jnp.float32),
            pltpu.VMEM((C,), jnp.int32),
            pltpu.VMEM((C,), jnp.int32),
            pltpu.VMEM((C, D), jnp.float32),
            pltpu.VMEM((C, D), jnp.float32),
            pltpu.VMEM((ZRB, D), jnp.float32),
            pltpu.VMEM((HR, D), jnp.float32),
            pltpu.VMEM((HR,), jnp.int32),
            pltpu.VMEM_SHARED((NN, D), jnp.float32),
            pltpu.VMEM_SHARED((HR, D), jnp.float32),
            pltpu.SemaphoreType.DMA,
            pltpu.SemaphoreType.DMA,
        ],
    )
    def k(h_hbm, ei_hbm, e_hbm, msg_out, cnt_out,
          src_b, dst_b, e_b, dst0, dst1, rows0, rows1,
          zrow_v, hist_v, hid_v, acc_sh, cnt_sh, sem0, sem1):
        cid = lax.axis_index("c")
        sid = lax.axis_index("s")
        wid = cid * NS + sid
        ones16 = jnp.ones((16,), jnp.float32)

        @pl.loop(0, ZRB)
        def _(r):
            for q in range(D // 16):
                zrow_v[r, pl.ds(q * 16, 16)] = jnp.zeros((16,), jnp.float32)

        @pl.loop(0, HR)
        def _(r):
            for q in range(D // 16):
                hist_v[r, pl.ds(q * 16, 16)] = jnp.zeros((16,), jnp.float32)

        for g in range(HR // 16):
            hid_v[pl.ds(g * 16, 16)] = lax.iota(jnp.int32, 16) + g * 16

        @pl.loop(0, NZB // NS)
        def _(t):
            j = t * NS + sid
            pltpu.sync_copy(zrow_v, acc_sh.at[pl.ds(j * ZRB, ZRB)])

        jrem = jnp.minimum((NZB // NS) * NS + sid, NZB - 1)
        pltpu.sync_copy(zrow_v, acc_sh.at[pl.ds(jrem * ZRB, ZRB)])
        jc = jnp.minimum(sid, HR // ZRB - 1)
        pltpu.sync_copy(zrow_v, cnt_sh.at[pl.ds(jc * ZRB, ZRB)])

        plsc.subcore_barrier()

        ebase = wid * EPW

        def gather(kk, rv, sem):
            pltpu.async_copy(h_hbm.at[src_b.at[pl.ds(kk * C, C)]], rv, sem)

        def gwait(rv, sem):
            pltpu.make_async_copy(h_hbm.at[src_b.at[pl.ds(0, C)]],
                                  rv, sem).wait()

        def process(kk, rv, dv):
            base = kk * C
            for g in range(C // 16):
                ivec = dst_b[pl.ds(base + g * 16, 16)]
                dv[pl.ds(g * 16, 16)] = ivec
                hrow = lax.shift_right_logical(ivec, 7)
                hcol = jnp.bitwise_and(ivec, 127)
                plsc.addupdate_scatter(hist_v, [hrow, hcol], ones16)

            @plsc.parallel_loop(0, C, 1, unroll=8)
            def _(r):
                ridx = jnp.full((16,), base + r, dtype=jnp.int32)
                s = plsc.load_gather(e_b, [ridx])
                for q in range(D // 16):
                    sl = pl.ds(q * 16, 16)
                    rv[r, sl] = rv[r, sl] * s

            pltpu.sync_copy(rv, acc_sh.at[dv], add=True)

        for b in range(NB):
            off = ebase + b * CB
            pltpu.sync_copy(ei_hbm.at[pl.ds(off, CB)], src_b)
            pltpu.sync_copy(ei_hbm.at[pl.ds(NE + off, CB)], dst_b)
            pltpu.sync_copy(e_hbm.at[pl.ds(off, CB)], e_b)

            gather(0, rows0, sem0)

            @pl.loop(0, (CPB - 1) // 2)
            def _(i):
                a = 2 * i
                gather(a + 1, rows1, sem1)
                gwait(rows0, sem0)
                process(a, rows0, dst0)
                gather(a + 2, rows0, sem0)
                gwait(rows1, sem1)
                process(a + 1, rows1, dst1)

            gwait(rows0, sem0)
            process(CPB - 1, rows0, dst0)

        pltpu.sync_copy(hist_v, cnt_sh.at[hid_v], add=True)
        plsc.subcore_barrier()

        for t in range(-(-NZCH // NS)):
            j = jnp.minimum(t * NS + sid, NZCH - 1)
            pltpu.sync_copy(acc_sh.at[pl.ds(j * ZR, ZR)],
                            msg_out.at[cid, pl.ds(j * ZR, ZR)])

        jw = jnp.minimum(sid, HR // ZRB - 1)
        pltpu.sync_copy(cnt_sh.at[pl.ds(jw * ZRB, ZRB)],
                        cnt_out.at[cid, pl.ds(jw * ZRB, ZRB)])

    return k(h, ei_flat, ev)


BR = 1000
_DN = (((1,), (1,)), ((), ()))


def _tc_body(h_ref, m0_ref, m1_ref, c0_ref, c1_ref, w_ref, b_ref, o_ref):
    cnt = c0_ref[...] + c1_ref[...]
    h_n = (m0_ref[...] + m1_ref[...]) / jnp.maximum(cnt, 1.0)
    o_ref[...] = (
        lax.dot_general(h_ref[...], w_ref[:, 0:D], _DN,
                        preferred_element_type=jnp.float32)
        + lax.dot_general(h_n, w_ref[:, D:2 * D], _DN,
                          preferred_element_type=jnp.float32)
        + b_ref[...]
    )


def _tc_combine(h, m0, m1, c0, c1, w, b2):
    return pl.pallas_call(
        _tc_body,
        grid=(NN // BR,),
        in_specs=[
            pl.BlockSpec((BR, D), lambda i: (i, 0)),
            pl.BlockSpec((BR, D), lambda i: (i, 0)),
            pl.BlockSpec((BR, D), lambda i: (i, 0)),
            pl.BlockSpec((BR, 1), lambda i: (i, 0)),
            pl.BlockSpec((BR, 1), lambda i: (i, 0)),
            pl.BlockSpec((D, 2 * D), lambda i: (0, 0)),
            pl.BlockSpec((1, D), lambda i: (0, 0)),
        ],
        out_specs=pl.BlockSpec((BR, D), lambda i: (i, 0)),
        out_shape=jax.ShapeDtypeStruct((NN, D), jnp.float32),
    )(h, m0, m1, c0, c1, w, b2)


def kernel(h, edge_index, e, W, b):
    ei_flat = edge_index.reshape(2 * NE)
    ev = e.reshape(NE)
    msg_p, cnt_p = _sc_aggregate(h, ei_flat, ev)
    cnt_flat = cnt_p.reshape(NC, HR * D)
    c0 = cnt_flat[0, :NN].reshape(NN, 1)
    c1 = cnt_flat[1, :NN].reshape(NN, 1)
    return _tc_combine(h, msg_p[0], msg_p[1], c0, c1, W, b.reshape(1, D))


# speedup vs baseline: 1.0477x; 1.0477x over previous
import dataclasses
import functools

import jax
import jax.numpy as jnp
from jax import lax
from jax.experimental import pallas as pl
from jax.experimental.pallas import tpu as pltpu
from jax.experimental.pallas import tpu_sc as plsc

NN = 10000
NE = 320000
D = 128
NC = 2
NS = 16
NW = NC * NS
EPW = NE // NW
C = 80
CB = 2000
NB = EPW // CB
CPB = CB // C
ZR = 40
NZCH = NN // ZR
ZRB = 8
NZB = NN // ZRB
HR = 80


def _mesh():
    return plsc.VectorSubcoreMesh(core_axis_name="c", subcore_axis_name="s")


def _compiler_params():
    cp = pltpu.CompilerParams()
    if "needs_layout_passes" in pltpu.CompilerParams.__dataclass_fields__:
        cp = dataclasses.replace(cp, needs_layout_passes=False)
    return cp


def _sc_aggregate(h, ei_flat, ev):
    @functools.partial(
        pl.kernel,
        compiler_params=_compiler_params(),
        out_type=[
            jax.ShapeDtypeStruct((NC, NN, D), jnp.float32),
            jax.ShapeDtypeStruct((NC, HR, D), jnp.float32),
        ],
        mesh=_mesh(),
        scratch_types=[
            pltpu.VMEM((CB,), jnp.int32),
            pltpu.VMEM((CB,), jnp.int32),
            pltpu.VMEM((CB,), jnp.float32),
            pltpu.VMEM((C,), jnp.int32),
            pltpu.VMEM((C,), jnp.int32),
            pltpu.VMEM((C, D), jnp.float32),
            pltpu.VMEM((C, D), jnp.float32),
            pltpu.VMEM((ZRB, D), jnp.float32),
            pltpu.VMEM((HR, D), jnp.float32),
            pltpu.VMEM((HR,), jnp.int32),
            pltpu.VMEM_SHARED((NN, D), jnp.float32),
            pltpu.VMEM_SHARED((HR, D), jnp.float32),
            pltpu.SemaphoreType.DMA,
            pltpu.SemaphoreType.DMA,
            pltpu.SemaphoreType.DMA,
        ],
    )
    def k(h_hbm, ei_hbm, e_hbm, msg_out, cnt_out,
          src_b, dst_b, e_b, dst0, dst1, rows0, rows1,
          zrow_v, hist_v, hid_v, acc_sh, cnt_sh, sem0, sem1, semz):
        cid = lax.axis_index("c")
        sid = lax.axis_index("s")
        wid = cid * NS + sid
        ones16 = jnp.ones((16,), jnp.float32)

        @pl.loop(0, ZRB)
        def _(r):
            for q in range(D // 16):
                zrow_v[r, pl.ds(q * 16, 16)] = jnp.zeros((16,), jnp.float32)

        @pl.loop(0, HR)
        def _(r):
            for q in range(D // 16):
                hist_v[r, pl.ds(q * 16, 16)] = jnp.zeros((16,), jnp.float32)

        for g in range(HR // 16):
            hid_v[pl.ds(g * 16, 16)] = lax.iota(jnp.int32, 16) + g * 16

        @pl.loop(0, NZB // NS)
        def _(t):
            j = t * NS + sid
            pltpu.async_copy(zrow_v, acc_sh.at[pl.ds(j * ZRB, ZRB)], semz)

        jrem = jnp.minimum((NZB // NS) * NS + sid, NZB - 1)
        pltpu.async_copy(zrow_v, acc_sh.at[pl.ds(jrem * ZRB, ZRB)], semz)
        jc = jnp.minimum(sid, HR // ZRB - 1)
        pltpu.async_copy(zrow_v, cnt_sh.at[pl.ds(jc * ZRB, ZRB)], semz)

        @pl.loop(0, NZB // NS + 2)
        def _(t):
            pltpu.make_async_copy(zrow_v, acc_sh.at[pl.ds(0, ZRB)],
                                  semz).wait()

        plsc.subcore_barrier()

        ebase = wid * EPW

        def gather(kk, rv, sem):
            pltpu.async_copy(h_hbm.at[src_b.at[pl.ds(kk * C, C)]], rv, sem)

        def gwait(rv, sem):
            pltpu.make_async_copy(h_hbm.at[src_b.at[pl.ds(0, C)]],
                                  rv, sem).wait()

        def process(kk, rv, dv):
            base = kk * C
            for g in range(C // 16):
                ivec = dst_b[pl.ds(base + g * 16, 16)]
                dv[pl.ds(g * 16, 16)] = ivec
                hrow = lax.shift_right_logical(ivec, 7)
                hcol = jnp.bitwise_and(ivec, 127)
                plsc.addupdate_scatter(hist_v, [hrow, hcol], ones16)

            @plsc.parallel_loop(0, C, 1, unroll=4)
            def _(r):
                ridx = jnp.full((16,), base + r, dtype=jnp.int32)
                s = plsc.load_gather(e_b, [ridx])
                for q in range(D // 16):
                    sl = pl.ds(q * 16, 16)
                    rv[r, sl] = rv[r, sl] * s

            pltpu.sync_copy(rv, acc_sh.at[dv], add=True)

        for b in range(NB):
            off = ebase + b * CB
            pltpu.sync_copy(ei_hbm.at[pl.ds(off, CB)], src_b)
            pltpu.sync_copy(ei_hbm.at[pl.ds(NE + off, CB)], dst_b)
            pltpu.sync_copy(e_hbm.at[pl.ds(off, CB)], e_b)

            gather(0, rows0, sem0)

            @pl.loop(0, (CPB - 1) // 2)
            def _(i):
                a = 2 * i
                gather(a + 1, rows1, sem1)
                gwait(rows0, sem0)
                process(a, rows0, dst0)
                gather(a + 2, rows0, sem0)
                gwait(rows1, sem1)
                process(a + 1, rows1, dst1)

            gwait(rows0, sem0)
            process(CPB - 1, rows0, dst0)

        pltpu.sync_copy(hist_v, cnt_sh.at[hid_v], add=True)
        plsc.subcore_barrier()

        for t in range(-(-NZCH // NS)):
            j = jnp.minimum(t * NS + sid, NZCH - 1)
            pltpu.async_copy(acc_sh.at[pl.ds(j * ZR, ZR)],
                             msg_out.at[cid, pl.ds(j * ZR, ZR)], semz)

        jw = jnp.minimum(sid, HR // ZRB - 1)
        pltpu.async_copy(cnt_sh.at[pl.ds(jw * ZRB, ZRB)],
                         cnt_out.at[cid, pl.ds(jw * ZRB, ZRB)], semz)

        for t in range(-(-NZCH // NS)):
            pltpu.make_async_copy(acc_sh.at[pl.ds(0, ZR)],
                                  msg_out.at[cid, pl.ds(0, ZR)], semz).wait()
        pltpu.make_async_copy(cnt_sh.at[pl.ds(0, ZRB)],
                              cnt_out.at[cid, pl.ds(0, ZRB)], semz).wait()

    return k(h, ei_flat, ev)


BR = 1000
_DN = (((1,), (1,)), ((), ()))


def _tc_body(h_ref, m0_ref, m1_ref, c0_ref, c1_ref, w_ref, b_ref, o_ref):
    cnt = c0_ref[...] + c1_ref[...]
    h_n = (m0_ref[...] + m1_ref[...]) / jnp.maximum(cnt, 1.0)
    o_ref[...] = (
        lax.dot_general(h_ref[...], w_ref[:, 0:D], _DN,
                        preferred_element_type=jnp.float32)
        + lax.dot_general(h_n, w_ref[:, D:2 * D], _DN,
                          preferred_element_type=jnp.float32)
        + b_ref[...]
    )


def _tc_combine(h, m0, m1, c0, c1, w, b2):
    return pl.pallas_call(
        _tc_body,
        grid=(NN // BR,),
        in_specs=[
            pl.BlockSpec((BR, D), lambda i: (i, 0)),
            pl.BlockSpec((BR, D), lambda i: (i, 0)),
            pl.BlockSpec((BR, D), lambda i: (i, 0)),
            pl.BlockSpec((BR, 1), lambda i: (i, 0)),
            pl.BlockSpec((BR, 1), lambda i: (i, 0)),
            pl.BlockSpec((D, 2 * D), lambda i: (0, 0)),
            pl.BlockSpec((1, D), lambda i: (0, 0)),
        ],
        out_specs=pl.BlockSpec((BR, D), lambda i: (i, 0)),
        out_shape=jax.ShapeDtypeStruct((NN, D), jnp.float32),
    )(h, m0, m1, c0, c1, w, b2)


def kernel(h, edge_index, e, W, b):
    ei_flat = edge_index.reshape(2 * NE)
    ev = e.reshape(NE)
    msg_p, cnt_p = _sc_aggregate(h, ei_flat, ev)
    cnt_flat = cnt_p.reshape(NC, HR * D)
    c0 = cnt_flat[0, :NN].reshape(NN, 1)
    c1 = cnt_flat[1, :NN].reshape(NN, 1)
    return _tc_combine(h, msg_p[0], msg_p[1], c0, c1, W, b.reshape(1, D))
